# baseline (device time: 53965 ns/iter reference)
import jax
import jax.numpy as jnp
from jax import lax
from jax.experimental import pallas as pl
from jax.experimental.pallas import tpu as pltpu

N_DEV = 8
ROWS = 4096
COLS = 512

_PARTS = (
    (0, 192, (4, 2, 1)),
    (1536, 176, (2, 1, 4)),
    (2944, 144, (1, 4, 2)),
)
_SOFF = ((0, 768, 1152), (1536, 2240, 2592), (2944, 3520, 3808))
_SCRATCH_ROWS = 4096
_READY_AFTER_BATCH = (1, 2, 3)


def kernel(x, k, Wp):
    b_sz, seq, c_loc = x.shape
    taps = k.shape[0]

    def body(x_hbm, k_ref, w_ref, out_hbm, xvm, pbuf, acc, scratch, staging,
             xsems, csems, send_sems, recv_sems):
        i = lax.axis_index("i")
        h = i ^ ((i >> 1) & 1)
        bit = {4: (h >> 2) & 1, 2: (h >> 1) & 1, 1: h & 1}

        def partner(mask):
            ph = h ^ mask
            return ph ^ ((ph >> 1) & 1)

        barrier = pltpu.get_barrier_semaphore()
        for m in (1, 2, 4):
            pl.semaphore_signal(
                barrier, inc=1,
                device_id=(partner(m),), device_id_type=pl.DeviceIdType.MESH,
            )

        w_bf = w_ref[...].astype(jnp.bfloat16)

        xvm[0, 0:8] = jnp.zeros((8, c_loc), jnp.float32)
        xvm[1, 0:8] = jnp.zeros((8, c_loc), jnp.float32)

        def xcopy(b):
            return pltpu.make_async_copy(
                x_hbm.at[b], xvm.at[b % 2, pl.ds(8, seq)], xsems.at[b % 2]
            )

        def compute_batch(b):
            slot = b % 2
            out = xvm[slot, 8 : 8 + seq] * k_ref[taps - 1 : taps, :]
            for t in range(taps - 1):
                d = taps - 1 - t
                out = out + xvm[slot, 8 - d : 8 - d + seq] * k_ref[t : t + 1, :]
            a = (out * jax.nn.sigmoid(out)).astype(jnp.bfloat16)
            pbuf[pl.ds(b * seq, seq)] = jnp.dot(
                a, w_bf, preferred_element_type=jnp.float32
            ).astype(jnp.bfloat16)

        def rs_rdma(p, s):
            B, K, order = _PARTS[p]
            a = order[s]
            prefix = B
            for j in range(s):
                prefix = prefix + bit[order[j]] * ((4 >> j) * K)
            size = (4 >> s) * K
            keep = prefix + bit[a] * size
            send = prefix + (1 - bit[a]) * size
            src = pbuf if s == 0 else acc
            rdma = pltpu.make_async_remote_copy(
                src_ref=src.at[pl.ds(send, size)],
                dst_ref=scratch.at[pl.ds(_SOFF[p][s], size)],
                send_sem=send_sems.at[6 * p + s],
                recv_sem=recv_sems.at[6 * p + s],
                device_id=(partner(a),),
                device_id_type=pl.DeviceIdType.MESH,
            )
            return rdma, keep, size

        def r2_range(p):
            B, K, order = _PARTS[p]
            return B + bit[order[0]] * (4 * K) + bit[order[1]] * (2 * K), 2 * K

        def exch_rdma(p):
            B, K, order = _PARTS[p]
            start, size = r2_range(p)
            rdma = pltpu.make_async_remote_copy(
                src_ref=acc.at[pl.ds(start, size)],
                dst_ref=scratch.at[pl.ds(_SOFF[p][2], size)],
                send_sem=send_sems.at[6 * p + 2],
                recv_sem=recv_sems.at[6 * p + 2],
                device_id=(partner(order[2]),),
                device_id_type=pl.DeviceIdType.MESH,
            )
            return rdma, start, size

        def ag_rdma(p, s):
            B, K, order = _PARTS[p]
            if s == 1:
                start, size = r2_range(p)
            else:
                start, size = B + bit[order[0]] * (4 * K), 4 * K
            return pltpu.make_async_remote_copy(
                src_ref=acc.at[pl.ds(start, size)],
                dst_ref=acc.at[pl.ds(start, size)],
                send_sem=send_sems.at[6 * p + 2 + s],
                recv_sem=recv_sems.at[6 * p + 2 + s],
                device_id=(partner(order[2 - s]),),
                device_id_type=pl.DeviceIdType.MESH,
            )

        xcopy(0).start()
        inflight = [None, None, None]
        next_part = 0
        for b in range(b_sz):
            if b + 1 < b_sz:
                xcopy(b + 1).start()
            xcopy(b).wait()
            compute_batch(b)
            if next_part < 3 and _READY_AFTER_BATCH[next_part] == b:
                if next_part == 0:
                    pl.semaphore_wait(barrier, 3)
                rdma, keep, size = rs_rdma(next_part, 0)
                rdma.start()
                inflight[next_part] = (rdma, keep, size)
                next_part += 1

        for s in range(2):
            nxt = [None, None, None]
            for p in range(3):
                rdma, keep, size = inflight[p]
                rdma.wait()
                base = pbuf if s == 0 else acc
                acc[pl.ds(keep, size)] = (
                    base[pl.ds(keep, size)]
                    + scratch[pl.ds(_SOFF[p][s], size)]
                )
                if s == 0:
                    r2, k2, z2 = rs_rdma(p, 1)
                    r2.start()
                    nxt[p] = (r2, k2, z2)
                else:
                    r2, k2, z2 = exch_rdma(p)
                    r2.start()
                    nxt[p] = (r2, k2, z2)
            inflight = nxt

        nxt = [None, None, None]
        for p in range(3):
            rdma, start, size = inflight[p]
            rdma.wait()
            acc[pl.ds(start, size)] = (
                acc[pl.ds(start, size)] + scratch[pl.ds(_SOFF[p][2], size)]
            )
            r2 = ag_rdma(p, 1)
            r2.start()
            nxt[p] = (r2, None, None)
        inflight = nxt

        nxt = [None, None, None]
        for p in range(3):
            inflight[p][0].wait()
            r2 = ag_rdma(p, 2)
            r2.start()
            nxt[p] = (r2, None, None)
        inflight = nxt

        copies = []
        for p in range(3):
            B, K, order = _PARTS[p]
            own = B + bit[order[0]] * (4 * K)
            staging[pl.ds(own, 4 * K)] = (
                acc[pl.ds(own, 4 * K)].astype(jnp.float32)
            )
            cp = pltpu.make_async_copy(
                staging.at[pl.ds(own, 4 * K)],
                out_hbm.at[pl.ds(own, 4 * K)],
                csems.at[2 * p],
            )
            cp.start()
            copies.append(cp)
        for p in range(3):
            inflight[p][0].wait()
            B, K, order = _PARTS[p]
            other = B + (1 - bit[order[0]]) * (4 * K)
            staging[pl.ds(other, 4 * K)] = (
                acc[pl.ds(other, 4 * K)].astype(jnp.float32)
            )
            cp = pltpu.make_async_copy(
                staging.at[pl.ds(other, 4 * K)],
                out_hbm.at[pl.ds(other, 4 * K)],
                csems.at[2 * p + 1],
            )
            cp.start()
            copies.append(cp)
        for cp in copies:
            cp.wait()

    reduced = pl.pallas_call(
        body,
        out_shape=jax.ShapeDtypeStruct((ROWS, COLS), jnp.float32),
        in_specs=[
            pl.BlockSpec(memory_space=pl.ANY),
            pl.BlockSpec(memory_space=pltpu.VMEM),
            pl.BlockSpec(memory_space=pltpu.VMEM),
        ],
        out_specs=pl.BlockSpec(memory_space=pl.ANY),
        scratch_shapes=[
            pltpu.VMEM((2, 1032, COLS), jnp.float32),
            pltpu.VMEM((ROWS, COLS), jnp.bfloat16),
            pltpu.VMEM((ROWS, COLS), jnp.bfloat16),
            pltpu.VMEM((_SCRATCH_ROWS, COLS), jnp.bfloat16),
            pltpu.VMEM((ROWS, COLS), jnp.float32),
            pltpu.SemaphoreType.DMA((2,)),
            pltpu.SemaphoreType.DMA((6,)),
            pltpu.SemaphoreType.DMA((18,)),
            pltpu.SemaphoreType.DMA((18,)),
        ],
        compiler_params=pltpu.CompilerParams(collective_id=0),
    )(x, k, Wp)
    return reduced.reshape(b_sz, seq, Wp.shape[1])


# device time: 52774 ns/iter; 1.0226x vs baseline; 1.0226x over previous
import jax
import jax.numpy as jnp
from jax import lax
from jax.experimental import pallas as pl
from jax.experimental.pallas import tpu as pltpu

N_DEV = 8
ROWS = 4096
COLS = 512

_PARTS = (
    (0, 176, (4, 2, 1)),
    (1408, 176, (2, 1, 4)),
    (2816, 160, (1, 4, 2)),
)
_SOFF = ((0, 704, 1056), (1408, 2112, 2464), (2816, 3456, 3776))
_SCRATCH_ROWS = 4096
_READY_AFTER_BATCH = (1, 2, 3)


def kernel(x, k, Wp):
    b_sz, seq, c_loc = x.shape
    taps = k.shape[0]

    def body(x_hbm, k_ref, w_ref, out_hbm, xvm, pbuf, acc, scratch, staging,
             xsems, csems, send_sems, recv_sems):
        i = lax.axis_index("i")
        h = i ^ ((i >> 1) & 1)
        bit = {4: (h >> 2) & 1, 2: (h >> 1) & 1, 1: h & 1}

        def partner(mask):
            ph = h ^ mask
            return ph ^ ((ph >> 1) & 1)

        barrier = pltpu.get_barrier_semaphore()
        for m in (1, 2, 4):
            pl.semaphore_signal(
                barrier, inc=1,
                device_id=(partner(m),), device_id_type=pl.DeviceIdType.MESH,
            )

        w_bf = w_ref[...].astype(jnp.bfloat16)

        xvm[0, 0:8] = jnp.zeros((8, c_loc), jnp.float32)
        xvm[1, 0:8] = jnp.zeros((8, c_loc), jnp.float32)

        def xcopy(b):
            return pltpu.make_async_copy(
                x_hbm.at[b], xvm.at[b % 2, pl.ds(8, seq)], xsems.at[b % 2]
            )

        def compute_batch(b):
            slot = b % 2
            out = xvm[slot, 8 : 8 + seq] * k_ref[taps - 1 : taps, :]
            for t in range(taps - 1):
                d = taps - 1 - t
                out = out + xvm[slot, 8 - d : 8 - d + seq] * k_ref[t : t + 1, :]
            a = (out * jax.nn.sigmoid(out)).astype(jnp.bfloat16)
            pbuf[pl.ds(b * seq, seq)] = jnp.dot(
                a, w_bf, preferred_element_type=jnp.float32
            ).astype(jnp.bfloat16)

        def rs_rdma(p, s):
            B, K, order = _PARTS[p]
            a = order[s]
            prefix = B
            for j in range(s):
                prefix = prefix + bit[order[j]] * ((4 >> j) * K)
            size = (4 >> s) * K
            keep = prefix + bit[a] * size
            send = prefix + (1 - bit[a]) * size
            src = pbuf if s == 0 else acc
            rdma = pltpu.make_async_remote_copy(
                src_ref=src.at[pl.ds(send, size)],
                dst_ref=scratch.at[pl.ds(_SOFF[p][s], size)],
                send_sem=send_sems.at[6 * p + s],
                recv_sem=recv_sems.at[6 * p + s],
                device_id=(partner(a),),
                device_id_type=pl.DeviceIdType.MESH,
            )
            return rdma, keep, size

        def r2_range(p):
            B, K, order = _PARTS[p]
            return B + bit[order[0]] * (4 * K) + bit[order[1]] * (2 * K), 2 * K

        def exch_rdma(p):
            B, K, order = _PARTS[p]
            start, size = r2_range(p)
            rdma = pltpu.make_async_remote_copy(
                src_ref=acc.at[pl.ds(start, size)],
                dst_ref=scratch.at[pl.ds(_SOFF[p][2], size)],
                send_sem=send_sems.at[6 * p + 2],
                recv_sem=recv_sems.at[6 * p + 2],
                device_id=(partner(order[2]),),
                device_id_type=pl.DeviceIdType.MESH,
            )
            return rdma, start, size

        def ag_rdma(p, s):
            B, K, order = _PARTS[p]
            if s == 1:
                start, size = r2_range(p)
            else:
                start, size = B + bit[order[0]] * (4 * K), 4 * K
            return pltpu.make_async_remote_copy(
                src_ref=acc.at[pl.ds(start, size)],
                dst_ref=acc.at[pl.ds(start, size)],
                send_sem=send_sems.at[6 * p + 2 + s],
                recv_sem=recv_sems.at[6 * p + 2 + s],
                device_id=(partner(order[2 - s]),),
                device_id_type=pl.DeviceIdType.MESH,
            )

        xcopy(0).start()
        inflight = [None, None, None]
        next_part = 0
        for b in range(b_sz):
            if b + 1 < b_sz:
                xcopy(b + 1).start()
            xcopy(b).wait()
            compute_batch(b)
            if next_part < 3 and _READY_AFTER_BATCH[next_part] == b:
                if next_part == 0:
                    pl.semaphore_wait(barrier, 3)
                rdma, keep, size = rs_rdma(next_part, 0)
                rdma.start()
                inflight[next_part] = (rdma, keep, size)
                next_part += 1

        for s in range(2):
            nxt = [None, None, None]
            for p in range(3):
                rdma, keep, size = inflight[p]
                rdma.wait()
                base = pbuf if s == 0 else acc
                acc[pl.ds(keep, size)] = (
                    base[pl.ds(keep, size)]
                    + scratch[pl.ds(_SOFF[p][s], size)]
                )
                if s == 0:
                    r2, k2, z2 = rs_rdma(p, 1)
                    r2.start()
                    nxt[p] = (r2, k2, z2)
                else:
                    r2, k2, z2 = exch_rdma(p)
                    r2.start()
                    nxt[p] = (r2, k2, z2)
            inflight = nxt

        nxt = [None, None, None]
        for p in range(3):
            rdma, start, size = inflight[p]
            rdma.wait()
            acc[pl.ds(start, size)] = (
                acc[pl.ds(start, size)] + scratch[pl.ds(_SOFF[p][2], size)]
            )
            r2 = ag_rdma(p, 1)
            r2.start()
            nxt[p] = (r2, None, None)
        inflight = nxt

        nxt = [None, None, None]
        for p in range(3):
            inflight[p][0].wait()
            r2 = ag_rdma(p, 2)
            r2.start()
            nxt[p] = (r2, None, None)
        inflight = nxt

        copies = []
        for p in range(3):
            B, K, order = _PARTS[p]
            own = B + bit[order[0]] * (4 * K)
            staging[pl.ds(own, 4 * K)] = (
                acc[pl.ds(own, 4 * K)].astype(jnp.float32)
            )
            cp = pltpu.make_async_copy(
                staging.at[pl.ds(own, 4 * K)],
                out_hbm.at[pl.ds(own, 4 * K)],
                csems.at[2 * p],
            )
            cp.start()
            copies.append(cp)
        for p in range(3):
            inflight[p][0].wait()
            B, K, order = _PARTS[p]
            other = B + (1 - bit[order[0]]) * (4 * K)
            staging[pl.ds(other, 4 * K)] = (
                acc[pl.ds(other, 4 * K)].astype(jnp.float32)
            )
            cp = pltpu.make_async_copy(
                staging.at[pl.ds(other, 4 * K)],
                out_hbm.at[pl.ds(other, 4 * K)],
                csems.at[2 * p + 1],
            )
            cp.start()
            copies.append(cp)
        for cp in copies:
            cp.wait()

    reduced = pl.pallas_call(
        body,
        out_shape=jax.ShapeDtypeStruct((ROWS, COLS), jnp.float32),
        in_specs=[
            pl.BlockSpec(memory_space=pl.ANY),
            pl.BlockSpec(memory_space=pltpu.VMEM),
            pl.BlockSpec(memory_space=pltpu.VMEM),
        ],
        out_specs=pl.BlockSpec(memory_space=pl.ANY),
        scratch_shapes=[
            pltpu.VMEM((2, 1032, COLS), jnp.float32),
            pltpu.VMEM((ROWS, COLS), jnp.bfloat16),
            pltpu.VMEM((ROWS, COLS), jnp.bfloat16),
            pltpu.VMEM((_SCRATCH_ROWS, COLS), jnp.bfloat16),
            pltpu.VMEM((ROWS, COLS), jnp.float32),
            pltpu.SemaphoreType.DMA((2,)),
            pltpu.SemaphoreType.DMA((6,)),
            pltpu.SemaphoreType.DMA((18,)),
            pltpu.SemaphoreType.DMA((18,)),
        ],
        compiler_params=pltpu.CompilerParams(collective_id=0),
    )(x, k, Wp)
    return reduced.reshape(b_sz, seq, Wp.shape[1])
